# baseline (speedup 1.0000x reference)
import numpy as np

import jax
import jax.numpy as jnp
from jax.experimental import pallas as pl
from jax.experimental.pallas import tpu as pltpu

B = 524288
D = 128
A = 21
AP = 32
TB = 32768


def _fold4(v, op):
    return op(op(v[0:8], v[8:16]), op(v[16:24], v[24:32]))


def _sum32(v):
    return jnp.sum(_fold4(v, jnp.add), axis=0, keepdims=True)


def _fused_kernel(x_ref, a_ref, wt_ref, b_ref, tau_ref, m0_ref,
                  sel_ref, ent_ref):
    xt = x_ref[...]
    lt = jnp.dot(xt, wt_ref[...], preferred_element_type=jnp.float32)
    l = lt.T + b_ref[...]
    ex = jnp.exp(l)
    s = _sum32(ex)
    rs = 1.0 / s
    u = _sum32(ex * l)
    ent = (jnp.log(s) - u * rs) * (1.0 / jnp.log(float(A)))

    two_a = a_ref[0] + a_ref[0]
    cmpf = jnp.where(two_a >= tau_ref[...], 1.0, 0.0)
    rolled = pltpu.roll(cmpf, 1, 0)
    onehot = jnp.where(m0_ref[...] > 0.0, 1.0 - cmpf, rolled - cmpf)
    sel = _sum32(onehot * ex) * rs

    sel_ref[0] = sel
    ent_ref[0] = ent


def _thresholds(action_values):
    lo, hi = action_values[:-1], action_values[1:]
    s = lo + hi
    e = hi - (s - lo)
    tau = jnp.where(e >= 0, jnp.nextafter(s, jnp.inf), s)
    out = jnp.full((AP, 1), jnp.inf, dtype=jnp.float32)
    return out.at[: A - 1, 0].set(tau)


def kernel(x, actions, W, b, action_values):
    nb = B // TB
    wt = jnp.zeros((D, AP), dtype=jnp.float32).at[:, :A].set(W.T)
    bp = jnp.full((AP, 1), -1e30, dtype=jnp.float32).at[:A, 0].set(b)
    tau = _thresholds(action_values)
    m0 = jnp.zeros((AP, 1), dtype=jnp.float32).at[0, 0].set(1.0)
    act3 = actions.reshape(nb, 1, TB)

    sel, ent = pl.pallas_call(
        _fused_kernel,
        grid=(nb,),
        in_specs=[
            pl.BlockSpec((TB, D), lambda i: (i, 0)),
            pl.BlockSpec((1, 1, TB), lambda i: (i, 0, 0)),
            pl.BlockSpec((D, AP), lambda i: (0, 0)),
            pl.BlockSpec((AP, 1), lambda i: (0, 0)),
            pl.BlockSpec((AP, 1), lambda i: (0, 0)),
            pl.BlockSpec((AP, 1), lambda i: (0, 0)),
        ],
        out_specs=[
            pl.BlockSpec((1, 1, TB), lambda i: (i, 0, 0)),
            pl.BlockSpec((1, 1, TB), lambda i: (i, 0, 0)),
        ],
        out_shape=[
            jax.ShapeDtypeStruct((nb, 1, TB), jnp.float32),
            jax.ShapeDtypeStruct((nb, 1, TB), jnp.float32),
        ],
        compiler_params=pltpu.CompilerParams(
            dimension_semantics=("parallel",),
        ),
    )(x, act3, wt, bp, tau, m0)
    return sel.reshape(B), ent.reshape(B)


# speedup vs baseline: 1.1890x; 1.1890x over previous
import jax
import jax.numpy as jnp
from jax.experimental import pallas as pl
from jax.experimental.pallas import tpu as pltpu

B = 524288
D = 128
A = 21
AP = 24
TB = 32768


def _folds(v, op):
    return op(op(v[0:8], v[8:16]), v[16:24])


def _sum_a(v):
    return jnp.sum(_folds(v, jnp.add), axis=0, keepdims=True)


def _min_a(v):
    return jnp.min(_folds(v, jnp.minimum), axis=0, keepdims=True)


def _fused_kernel(x_ref, a_ref, wt_ref, b_ref, av_ref, sel_ref, ent_ref):
    xt = x_ref[...]
    lt = jnp.dot(xt, wt_ref[...], preferred_element_type=jnp.float32)
    l = lt.T + b_ref[...]
    ex = jnp.exp(l)
    s = _sum_a(ex)
    rs = 1.0 / s
    u = _sum_a(ex * l)
    ent = (jnp.log(s) - u * rs) * (1.0 / jnp.log(float(A)))

    act = a_ref[0]
    diffs = jnp.abs(act - av_ref[...])
    mind = _min_a(diffs)
    iota = jax.lax.broadcasted_iota(jnp.int32, (AP, TB), 0)
    idx = _min_a(jnp.where(diffs == mind, iota, AP))
    sel = _sum_a(jnp.where(iota == idx, ex, 0.0)) * rs

    sel_ref[0] = sel
    ent_ref[0] = ent


def kernel(x, actions, W, b, action_values):
    nb = B // TB
    wt = jnp.zeros((D, AP), dtype=jnp.float32).at[:, :A].set(W.T)
    bp = jnp.full((AP, 1), -1e30, dtype=jnp.float32).at[:A, 0].set(b)
    avp = jnp.full((AP, 1), 1e30, dtype=jnp.float32).at[:A, 0].set(action_values)
    act3 = actions.reshape(nb, 1, TB)

    sel, ent = pl.pallas_call(
        _fused_kernel,
        grid=(nb,),
        in_specs=[
            pl.BlockSpec((TB, D), lambda i: (i, 0)),
            pl.BlockSpec((1, 1, TB), lambda i: (i, 0, 0)),
            pl.BlockSpec((D, AP), lambda i: (0, 0)),
            pl.BlockSpec((AP, 1), lambda i: (0, 0)),
            pl.BlockSpec((AP, 1), lambda i: (0, 0)),
        ],
        out_specs=[
            pl.BlockSpec((1, 1, TB), lambda i: (i, 0, 0)),
            pl.BlockSpec((1, 1, TB), lambda i: (i, 0, 0)),
        ],
        out_shape=[
            jax.ShapeDtypeStruct((nb, 1, TB), jnp.float32),
            jax.ShapeDtypeStruct((nb, 1, TB), jnp.float32),
        ],
        compiler_params=pltpu.CompilerParams(
            dimension_semantics=("parallel",),
        ),
    )(x, act3, wt, bp, avp)
    return sel.reshape(B), ent.reshape(B)
